# Initial kernel scaffold; baseline (speedup 1.0000x reference)
import jax
import jax.numpy as jnp
from jax.experimental import pallas as pl


def kernel(coordinates, edge_attr, src, dst, non_fictitious, non_fictitious_addresses, params):
    raise NotImplementedError("write your pallas kernel here")


# speedup vs baseline: 2.0827x; 2.0827x over previous
import functools

import jax
import jax.numpy as jnp
from jax import lax
from jax.experimental import pallas as pl
from jax.experimental.pallas import tpu as pltpu
from jax.experimental.pallas import tpu_sc as plsc

NC = 2
NS = 16
NW = NC * NS
C = 80
EB = 800
NB = 2000

_VNAMES = ["value_src_0", "value_src_1", "value_dst_0", "value_dst_1"]
_SNAMES = ["score_src_0", "score_src_1", "score_dst_0", "score_dst_1"]


def _mesh():
    return plsc.VectorSubcoreMesh(core_axis_name="c", subcore_axis_name="s")


def _make_gather(n_nodes, d_feat, n_edges):
    epw = n_edges // NW
    nch = epw // C

    @functools.partial(
        pl.kernel,
        mesh=_mesh(),
        out_type=[
            jax.ShapeDtypeStruct((n_edges, d_feat), jnp.float32),
            jax.ShapeDtypeStruct((n_edges, d_feat), jnp.float32),
        ],
        scratch_types=[
            pltpu.VMEM((epw,), jnp.int32),
            pltpu.VMEM((epw,), jnp.int32),
            pltpu.VMEM((C, d_feat), jnp.float32),
            pltpu.VMEM((C, d_feat), jnp.float32),
            pltpu.SemaphoreType.DMA,
            pltpu.SemaphoreType.DMA,
        ],
    )
    def gather_k(coords, srcv, dstv, gs, gd, sidx, didx, rs, rd, sem_s, sem_d):
        cid = lax.axis_index("c")
        sid = lax.axis_index("s")
        wid = sid * NC + cid
        base = pl.multiple_of(wid * epw, 8)
        pltpu.sync_copy(srcv.at[pl.ds(base, epw)], sidx)
        pltpu.sync_copy(dstv.at[pl.ds(base, epw)], didx)

        def body(j, carry):
            off = pl.multiple_of(j * C, 8)
            cp1 = pltpu.async_copy(coords.at[sidx.at[pl.ds(off, C)]], rs, sem_s)
            cp2 = pltpu.async_copy(coords.at[didx.at[pl.ds(off, C)]], rd, sem_d)
            cp1.wait()
            cp2.wait()
            pltpu.sync_copy(rs, gs.at[pl.ds(base + off, C)])
            pltpu.sync_copy(rd, gd.at[pl.ds(base + off, C)])
            return carry

        lax.fori_loop(0, nch, body, 0)

    return gather_k


def _make_scatter(n_nodes, n_edges):
    epw = n_edges // NS
    nph = 10
    nch = epw // (C * nph)
    nd = n_nodes // 8

    @functools.partial(
        pl.kernel,
        mesh=plsc.VectorSubcoreMesh(
            core_axis_name="c", subcore_axis_name="s", num_cores=1),
        out_type=[
            jax.ShapeDtypeStruct((n_nodes, 128), jnp.float32),
            jax.ShapeDtypeStruct((nd, 128), jnp.float32),
        ],
        scratch_types=[
            pltpu.VMEM((nch, C), jnp.int32),
            pltpu.VMEM((nch, C), jnp.int32),
            pltpu.VMEM((nch, C), jnp.int32),
            pltpu.VMEM((nch, C), jnp.int32),
            pltpu.VMEM((C, 128), jnp.float32),
            pltpu.VMEM_SHARED((n_nodes, 128), jnp.float32),
            pltpu.VMEM_SHARED((nd, 128), jnp.float32),
        ],
    )
    def scatter_k(ps, pd, dps, dpd, sidx4, didx4, sidx84, didx84, zn, zd,
                  outn, outd, sidx, didx, sidx8, didx8,
                  rbuf, accn, accd):
        sid = lax.axis_index("s")

        @pl.when(sid == 0)
        def _():
            pltpu.sync_copy(zn, accn)
            pltpu.sync_copy(zd, accd)

        plsc.subcore_barrier()

        def phase(p, carry):
            pltpu.sync_copy(sidx4.at[sid, p], sidx)
            pltpu.sync_copy(didx4.at[sid, p], didx)
            pltpu.sync_copy(sidx84.at[sid, p], sidx8)
            pltpu.sync_copy(didx84.at[sid, p], didx8)
            base = pl.multiple_of(sid * epw + p * nch * C, 8)

            def body(j, carry2):
                off = pl.multiple_of(j * C, 8)
                pltpu.sync_copy(ps.at[pl.ds(base + off, C)], rbuf)
                pltpu.sync_copy(rbuf, accn.at[sidx.at[j]], add=True)
                pltpu.sync_copy(pd.at[pl.ds(base + off, C)], rbuf)
                pltpu.sync_copy(rbuf, accn.at[didx.at[j]], add=True)
                pltpu.sync_copy(dps.at[pl.ds(base + off, C)], rbuf)
                pltpu.sync_copy(rbuf, accd.at[sidx8.at[j]], add=True)
                pltpu.sync_copy(dpd.at[pl.ds(base + off, C)], rbuf)
                pltpu.sync_copy(rbuf, accd.at[didx8.at[j]], add=True)
                return carry2

            lax.fori_loop(0, nch, body, 0)
            return carry

        lax.fori_loop(0, nph, phase, 0)
        plsc.subcore_barrier()

        @pl.when(sid == 0)
        def _():
            pltpu.sync_copy(accn, outn)
            pltpu.sync_copy(accd, outd)

    return scatter_k


def _edge_body(gs_ref, gd_ref, ea_ref, nf_ref, src_ref, dst_ref,
               w1e_ref, w1s_ref, w1d_ref, b1_ref, w2v_ref, b2v_ref,
               w2s_ref, b2s_ref, ps_ref, pd_ref, dps_ref, dpd_ref):
    x = jnp.dot(ea_ref[...], w1e_ref[...], preferred_element_type=jnp.float32)
    x = x + jnp.dot(gs_ref[...], w1s_ref[...], preferred_element_type=jnp.float32)
    x = x + jnp.dot(gd_ref[...], w1d_ref[...], preferred_element_type=jnp.float32)
    h = jnp.maximum(x + b1_ref[...], 0.0)
    nf = nf_ref[...]
    v = (jnp.dot(h[:, :256], w2v_ref[...], preferred_element_type=jnp.float32)
         + b2v_ref[...]) * nf
    s = (jnp.dot(h[:, 256:], w2s_ref[...], preferred_element_type=jnp.float32)
         + b2s_ref[...]) * nf
    e = jnp.exp(s)
    b = v.shape[0]
    ps_ref[...] = jnp.concatenate(
        [v[:, 0:64] * e[:, 0:1], v[:, 64:128] * e[:, 1:2]], axis=1)
    pd_ref[...] = jnp.concatenate(
        [v[:, 128:192] * e[:, 2:3], v[:, 192:256] * e[:, 3:4]], axis=1)
    lane = lax.broadcasted_iota(jnp.int32, (b, 128), 1)
    scol = (src_ref[...] % 8) * 16
    dcol = (dst_ref[...] % 8) * 16
    zero = jnp.zeros((b, 128), jnp.float32)
    dps_ref[...] = (jnp.where(lane == scol, e[:, 0:1], zero)
                    + jnp.where(lane == scol + 8, e[:, 1:2], zero))
    dpd_ref[...] = (jnp.where(lane == dcol, e[:, 2:3], zero)
                    + jnp.where(lane == dcol + 8, e[:, 3:4], zero))


def _run_edge_mlp(gs, gd, ea, nf2, src2, dst2,
                  w1e, w1s, w1d, b1, w2v, b2v, w2s, b2s):
    n_edges, d_feat = gs.shape
    d_edge = ea.shape[1]
    nblk = n_edges // EB
    full = lambda shp: pl.BlockSpec(shp, lambda i: (0, 0))
    row = lambda w: pl.BlockSpec((EB, w), lambda i: (i, 0))
    return pl.pallas_call(
        _edge_body,
        grid=(nblk,),
        in_specs=[
            row(d_feat), row(d_feat), row(d_edge), row(1), row(1), row(1),
            full((d_edge, 512)), full((d_feat, 512)), full((d_feat, 512)),
            full((1, 512)), full((256, 256)), full((1, 256)),
            full((256, 8)), full((1, 8)),
        ],
        out_specs=[row(128), row(128), row(128), row(128)],
        out_shape=[jax.ShapeDtypeStruct((n_edges, 128), jnp.float32)] * 4,
    )(gs, gd, ea, nf2, src2, dst2, w1e, w1s, w1d, b1, w2v, b2v, w2s, b2s)


def _psi_body(accn_ref, den_ref, nfa_ref, p1_ref, pb1_ref, p2_ref, pb2_ref,
              out_ref):
    a = accn_ref[...]
    den = den_ref[...]
    nfa = nfa_ref[...]
    num = a * nfa
    d0 = den[:, 0:1] * nfa + 1e-9
    d1 = den[:, 8:9] * nfa + 1e-9
    vp = jnp.concatenate([num[:, 0:64] / d0, num[:, 64:128] / d1], axis=1)
    h = jnp.maximum(
        jnp.dot(vp, p1_ref[...], preferred_element_type=jnp.float32) + pb1_ref[...],
        0.0)
    out_ref[...] = (jnp.dot(h, p2_ref[...], preferred_element_type=jnp.float32)
                    + pb2_ref[...]) * nfa


def _run_psi(accn, den, nfa2, p1, pb1, p2, pb2):
    n_nodes = accn.shape[0]
    d_out = p2.shape[1]
    d_in = p1.shape[0]
    nblk = n_nodes // NB
    full = lambda shp: pl.BlockSpec(shp, lambda i: (0, 0))
    return pl.pallas_call(
        _psi_body,
        grid=(nblk,),
        in_specs=[
            pl.BlockSpec((NB, 128), lambda i: (i, 0)),
            pl.BlockSpec((NB, 16), lambda i: (i, 0)),
            pl.BlockSpec((NB, 1), lambda i: (i, 0)),
            full((d_in, d_in)), full((1, d_in)), full((d_in, d_out)),
            full((1, d_out)),
        ],
        out_specs=pl.BlockSpec((NB, d_out), lambda i: (i, 0)),
        out_shape=jax.ShapeDtypeStruct((n_nodes, d_out), jnp.float32),
    )(accn, den, nfa2, p1, pb1, p2, pb2)


def kernel(coordinates, edge_attr, src, dst, non_fictitious,
           non_fictitious_addresses, params):
    n_nodes, d_feat = coordinates.shape
    n_edges, d_edge = edge_attr.shape
    epw = n_edges // NW
    nch = epw // C

    w1 = jnp.concatenate([params[n][0][0] for n in _VNAMES + _SNAMES], axis=1)
    b1 = jnp.concatenate([params[n][0][1] for n in _VNAMES + _SNAMES])[None, :]
    w1e = w1[:d_edge]
    w1s = w1[d_edge:d_edge + d_feat]
    w1d = w1[d_edge + d_feat:]
    w2v = jnp.zeros((256, 256), jnp.float32)
    for m, n in enumerate(_VNAMES):
        w2v = w2v.at[m * 64:(m + 1) * 64, m * 64:(m + 1) * 64].set(params[n][1][0])
    b2v = jnp.concatenate([params[n][1][1] for n in _VNAMES])[None, :]
    w2s = jnp.zeros((256, 8), jnp.float32)
    for m, n in enumerate(_SNAMES):
        w2s = w2s.at[m * 64:(m + 1) * 64, m].set(params[n][1][0][:, 0])
    b2s = jnp.zeros((1, 8), jnp.float32).at[0, :4].set(
        jnp.stack([params[n][1][1][0] for n in _SNAMES]))
    p1, pb1 = params["psi"][0]
    p2, pb2 = params["psi"][1]

    gs, gd = _make_gather(n_nodes, d_feat, n_edges)(coordinates, src, dst)
    ps, pd, dps, dpd = _run_edge_mlp(
        gs, gd, edge_attr, non_fictitious[:, None],
        src[:, None], dst[:, None],
        w1e, w1s, w1d, b1, w2v, b2v, w2s, b2s)
    zn = jnp.zeros((n_nodes, 128), jnp.float32)
    zd = jnp.zeros((n_nodes // 8, 128), jnp.float32)
    nch_s = (n_edges // NS) // (C * 10)
    ishape = (NS, 10, nch_s, C)
    accn, accd = _make_scatter(n_nodes, n_edges)(
        ps, pd, dps, dpd,
        src.reshape(ishape), dst.reshape(ishape),
        (src // 8).reshape(ishape), (dst // 8).reshape(ishape),
        zn, zd)
    den = accd.reshape(n_nodes, 16)
    return _run_psi(accn, den, non_fictitious_addresses[:, None],
                    p1, pb1[None, :], p2, pb2[None, :])


# speedup vs baseline: 3.3087x; 1.5887x over previous
import functools

import jax
import jax.numpy as jnp
from jax import lax
from jax.experimental import pallas as pl
from jax.experimental.pallas import tpu as pltpu
from jax.experimental.pallas import tpu_sc as plsc

NC = 2
NS = 16
NW = NC * NS
C = 80
EB = 800
NB = 2000

_VNAMES = ["value_src_0", "value_src_1", "value_dst_0", "value_dst_1"]
_SNAMES = ["score_src_0", "score_src_1", "score_dst_0", "score_dst_1"]


def _mesh():
    return plsc.VectorSubcoreMesh(core_axis_name="c", subcore_axis_name="s")


NBUF_G = 5


def _make_gather(n_nodes, d_feat, n_edges):
    epw = n_edges // NW
    nch = epw // C
    ngrp = nch // NBUF_G

    scratch = [pltpu.VMEM((epw,), jnp.int32), pltpu.VMEM((epw,), jnp.int32)]
    scratch += [pltpu.VMEM((C, d_feat), jnp.float32) for _ in range(2 * NBUF_G)]
    scratch += [pltpu.SemaphoreType.DMA for _ in range(4 * NBUF_G)]

    @functools.partial(
        pl.kernel,
        mesh=_mesh(),
        out_type=[
            jax.ShapeDtypeStruct((n_edges, d_feat), jnp.float32),
            jax.ShapeDtypeStruct((n_edges, d_feat), jnp.float32),
        ],
        scratch_types=scratch,
    )
    def gather_k(coords, srcv, dstv, gs, gd, *scr):
        sidx, didx = scr[0], scr[1]
        rs = scr[2:2 + NBUF_G]
        rd = scr[2 + NBUF_G:2 + 2 * NBUF_G]
        base_sem = 2 + 2 * NBUF_G
        gsem_s = scr[base_sem:base_sem + NBUF_G]
        gsem_d = scr[base_sem + NBUF_G:base_sem + 2 * NBUF_G]
        wsem_s = scr[base_sem + 2 * NBUF_G:base_sem + 3 * NBUF_G]
        wsem_d = scr[base_sem + 3 * NBUF_G:base_sem + 4 * NBUF_G]

        cid = lax.axis_index("c")
        sid = lax.axis_index("s")
        wid = sid * NC + cid
        base = pl.multiple_of(wid * epw, 8)
        pltpu.sync_copy(srcv.at[pl.ds(base, epw)], sidx)
        pltpu.sync_copy(dstv.at[pl.ds(base, epw)], didx)

        def fire(ch, b):
            off = pl.multiple_of(ch * C, 8)
            pltpu.async_copy(coords.at[sidx.at[pl.ds(off, C)]], rs[b], gsem_s[b])
            pltpu.async_copy(coords.at[didx.at[pl.ds(off, C)]], rd[b], gsem_d[b])

        for b in range(NBUF_G):
            fire(b, b)

        def grp(g, carry):
            for b in range(NBUF_G):
                ch = g * NBUF_G + b
                off = pl.multiple_of(ch * C, 8)
                pltpu.make_async_copy(coords.at[pl.ds(0, C)], rs[b], gsem_s[b]).wait()
                pltpu.make_async_copy(coords.at[pl.ds(0, C)], rd[b], gsem_d[b]).wait()
                pltpu.async_copy(rs[b], gs.at[pl.ds(base + off, C)], wsem_s[b])
                pltpu.async_copy(rd[b], gd.at[pl.ds(base + off, C)], wsem_d[b])
                pltpu.make_async_copy(rs[b], gs.at[pl.ds(base, C)], wsem_s[b]).wait()
                pltpu.make_async_copy(rd[b], gd.at[pl.ds(base, C)], wsem_d[b]).wait()
                nxt = ch + NBUF_G

                @pl.when(nxt < nch)
                def _():
                    fire(nxt, b)
            return carry

        lax.fori_loop(0, ngrp, grp, 0)

    return gather_k


NPH_S = 5
NBUF_S = 2


def _make_scatter(n_nodes, n_edges):
    epw = n_edges // NS
    nch = epw // C
    npc = nch // NPH_S
    nd = n_nodes // 8

    scratch = [pltpu.VMEM((npc, C), jnp.int32)]
    scratch += [pltpu.VMEM((C, 128), jnp.float32) for _ in range(NBUF_S)]
    scratch += [pltpu.VMEM_SHARED((n_nodes, 128), jnp.float32),
                pltpu.VMEM_SHARED((nd, 128), jnp.float32)]
    scratch += [pltpu.SemaphoreType.DMA for _ in range(2 * NBUF_S)]

    @functools.partial(
        pl.kernel,
        mesh=_mesh(),
        out_type=[
            jax.ShapeDtypeStruct((n_nodes, 128), jnp.float32),
            jax.ShapeDtypeStruct((nd, 128), jnp.float32),
        ],
        scratch_types=scratch,
    )
    def scatter_k(ps, pd, dps, dpd, sidx4, didx4, sidx84, didx84, zn, zd,
                  outn, outd, *scr):
        idxb = scr[0]
        rb = scr[1:1 + NBUF_S]
        accn, accd = scr[1 + NBUF_S], scr[2 + NBUF_S]
        lsem = scr[3 + NBUF_S:3 + 2 * NBUF_S]
        ssem = scr[3 + 2 * NBUF_S:3 + 3 * NBUF_S]
        cid = lax.axis_index("c")
        sid = lax.axis_index("s")

        @pl.when(sid == 0)
        def _():
            pltpu.sync_copy(zn, accn)
            pltpu.sync_copy(zd, accd)

        plsc.subcore_barrier()

        def do_pass(payload, idx4, acc):
            def phase(p, carry):
                pltpu.sync_copy(idx4.at[sid, p], idxb)
                pbase = pl.multiple_of(sid * epw + p * npc * C, 8)

                def load(ch, b):
                    off = pl.multiple_of(ch * C, 8)
                    pltpu.async_copy(
                        payload.at[pl.ds(pbase + off, C)], rb[b], lsem[b])

                for b in range(NBUF_S):
                    load(b, b)

                def grp(g, carry2):
                    for b in range(NBUF_S):
                        ch = g * NBUF_S + b
                        pltpu.make_async_copy(
                            payload.at[pl.ds(pbase, C)], rb[b], lsem[b]).wait()
                        pltpu.async_copy(
                            rb[b], acc.at[idxb.at[ch]], ssem[b], add=True)
                        pltpu.make_async_copy(
                            rb[b], acc.at[idxb.at[ch]], ssem[b]).wait()
                        nxt = ch + NBUF_S

                        @pl.when(nxt < npc)
                        def _():
                            load(nxt, b)
                    return carry2

                lax.fori_loop(0, npc // NBUF_S, grp, 0)
                return carry

            lax.fori_loop(0, NPH_S, phase, 0)

        @pl.when(cid == 0)
        def _():
            do_pass(ps, sidx4, accn)
            do_pass(pd, didx4, accn)

        @pl.when(cid == 1)
        def _():
            do_pass(dps, sidx84, accd)
            do_pass(dpd, didx84, accd)

        plsc.subcore_barrier()

        @pl.when((sid == 0) & (cid == 0))
        def _():
            pltpu.sync_copy(accn, outn)

        @pl.when((sid == 0) & (cid == 1))
        def _():
            pltpu.sync_copy(accd, outd)

    return scatter_k


def _edge_body(gs_ref, gd_ref, ea_ref, nf_ref, src_ref, dst_ref,
               w1e_ref, w1s_ref, w1d_ref, b1_ref, w2v_ref, b2v_ref,
               w2s_ref, b2s_ref, ps_ref, pd_ref, dps_ref, dpd_ref):
    x = jnp.dot(ea_ref[...], w1e_ref[...], preferred_element_type=jnp.float32)
    x = x + jnp.dot(gs_ref[...], w1s_ref[...], preferred_element_type=jnp.float32)
    x = x + jnp.dot(gd_ref[...], w1d_ref[...], preferred_element_type=jnp.float32)
    h = jnp.maximum(x + b1_ref[...], 0.0)
    nf = nf_ref[...]
    v = (jnp.dot(h[:, :256], w2v_ref[...], preferred_element_type=jnp.float32)
         + b2v_ref[...]) * nf
    s = (jnp.dot(h[:, 256:], w2s_ref[...], preferred_element_type=jnp.float32)
         + b2s_ref[...]) * nf
    e = jnp.exp(s)
    b = v.shape[0]
    ps_ref[...] = jnp.concatenate(
        [v[:, 0:64] * e[:, 0:1], v[:, 64:128] * e[:, 1:2]], axis=1)
    pd_ref[...] = jnp.concatenate(
        [v[:, 128:192] * e[:, 2:3], v[:, 192:256] * e[:, 3:4]], axis=1)
    lane = lax.broadcasted_iota(jnp.int32, (b, 128), 1)
    scol = (src_ref[...] % 8) * 16
    dcol = (dst_ref[...] % 8) * 16
    zero = jnp.zeros((b, 128), jnp.float32)
    dps_ref[...] = (jnp.where(lane == scol, e[:, 0:1], zero)
                    + jnp.where(lane == scol + 8, e[:, 1:2], zero))
    dpd_ref[...] = (jnp.where(lane == dcol, e[:, 2:3], zero)
                    + jnp.where(lane == dcol + 8, e[:, 3:4], zero))


def _run_edge_mlp(gs, gd, ea, nf2, src2, dst2,
                  w1e, w1s, w1d, b1, w2v, b2v, w2s, b2s):
    n_edges, d_feat = gs.shape
    d_edge = ea.shape[1]
    nblk = n_edges // EB
    full = lambda shp: pl.BlockSpec(shp, lambda i: (0, 0))
    row = lambda w: pl.BlockSpec((EB, w), lambda i: (i, 0))
    return pl.pallas_call(
        _edge_body,
        grid=(nblk,),
        in_specs=[
            row(d_feat), row(d_feat), row(d_edge), row(1), row(1), row(1),
            full((d_edge, 512)), full((d_feat, 512)), full((d_feat, 512)),
            full((1, 512)), full((256, 256)), full((1, 256)),
            full((256, 8)), full((1, 8)),
        ],
        out_specs=[row(128), row(128), row(128), row(128)],
        out_shape=[jax.ShapeDtypeStruct((n_edges, 128), jnp.float32)] * 4,
    )(gs, gd, ea, nf2, src2, dst2, w1e, w1s, w1d, b1, w2v, b2v, w2s, b2s)


def _psi_body(accn_ref, den_ref, nfa_ref, p1_ref, pb1_ref, p2_ref, pb2_ref,
              out_ref):
    a = accn_ref[...]
    den = den_ref[...]
    nfa = nfa_ref[...]
    num = a * nfa
    d0 = den[:, 0:1] * nfa + 1e-9
    d1 = den[:, 8:9] * nfa + 1e-9
    vp = jnp.concatenate([num[:, 0:64] / d0, num[:, 64:128] / d1], axis=1)
    h = jnp.maximum(
        jnp.dot(vp, p1_ref[...], preferred_element_type=jnp.float32) + pb1_ref[...],
        0.0)
    out_ref[...] = (jnp.dot(h, p2_ref[...], preferred_element_type=jnp.float32)
                    + pb2_ref[...]) * nfa


def _run_psi(accn, den, nfa2, p1, pb1, p2, pb2):
    n_nodes = accn.shape[0]
    d_out = p2.shape[1]
    d_in = p1.shape[0]
    nblk = n_nodes // NB
    full = lambda shp: pl.BlockSpec(shp, lambda i: (0, 0))
    return pl.pallas_call(
        _psi_body,
        grid=(nblk,),
        in_specs=[
            pl.BlockSpec((NB, 128), lambda i: (i, 0)),
            pl.BlockSpec((NB, 16), lambda i: (i, 0)),
            pl.BlockSpec((NB, 1), lambda i: (i, 0)),
            full((d_in, d_in)), full((1, d_in)), full((d_in, d_out)),
            full((1, d_out)),
        ],
        out_specs=pl.BlockSpec((NB, d_out), lambda i: (i, 0)),
        out_shape=jax.ShapeDtypeStruct((n_nodes, d_out), jnp.float32),
    )(accn, den, nfa2, p1, pb1, p2, pb2)


def kernel(coordinates, edge_attr, src, dst, non_fictitious,
           non_fictitious_addresses, params):
    n_nodes, d_feat = coordinates.shape
    n_edges, d_edge = edge_attr.shape
    epw = n_edges // NW
    nch = epw // C

    w1 = jnp.concatenate([params[n][0][0] for n in _VNAMES + _SNAMES], axis=1)
    b1 = jnp.concatenate([params[n][0][1] for n in _VNAMES + _SNAMES])[None, :]
    w1e = w1[:d_edge]
    w1s = w1[d_edge:d_edge + d_feat]
    w1d = w1[d_edge + d_feat:]
    w2v = jnp.zeros((256, 256), jnp.float32)
    for m, n in enumerate(_VNAMES):
        w2v = w2v.at[m * 64:(m + 1) * 64, m * 64:(m + 1) * 64].set(params[n][1][0])
    b2v = jnp.concatenate([params[n][1][1] for n in _VNAMES])[None, :]
    w2s = jnp.zeros((256, 8), jnp.float32)
    for m, n in enumerate(_SNAMES):
        w2s = w2s.at[m * 64:(m + 1) * 64, m].set(params[n][1][0][:, 0])
    b2s = jnp.zeros((1, 8), jnp.float32).at[0, :4].set(
        jnp.stack([params[n][1][1][0] for n in _SNAMES]))
    p1, pb1 = params["psi"][0]
    p2, pb2 = params["psi"][1]

    gs, gd = _make_gather(n_nodes, d_feat, n_edges)(coordinates, src, dst)
    ps, pd, dps, dpd = _run_edge_mlp(
        gs, gd, edge_attr, non_fictitious[:, None],
        src[:, None], dst[:, None],
        w1e, w1s, w1d, b1, w2v, b2v, w2s, b2s)
    zn = jnp.zeros((n_nodes, 128), jnp.float32)
    zd = jnp.zeros((n_nodes // 8, 128), jnp.float32)
    nch_s = (n_edges // NS) // (C * NPH_S)
    ishape = (NS, NPH_S, nch_s, C)
    accn, accd = _make_scatter(n_nodes, n_edges)(
        ps, pd, dps, dpd,
        src.reshape(ishape), dst.reshape(ishape),
        (src // 8).reshape(ishape), (dst // 8).reshape(ishape),
        zn, zd)
    den = accd.reshape(n_nodes, 16)
    return _run_psi(accn, den, non_fictitious_addresses[:, None],
                    p1, pb1[None, :], p2, pb2[None, :])


# speedup vs baseline: 3.6732x; 1.1101x over previous
import functools

import jax
import jax.numpy as jnp
from jax import lax
from jax.experimental import pallas as pl
from jax.experimental.pallas import tpu as pltpu
from jax.experimental.pallas import tpu_sc as plsc

NC = 2
NS = 16
NW = NC * NS
C = 80
K = 5
EB = 800
NB = 2000
NDPAD = 1280

_VNAMES = ["value_src_0", "value_src_1", "value_dst_0", "value_dst_1"]
_SNAMES = ["score_src_0", "score_src_1", "score_dst_0", "score_dst_1"]


def _mesh():
    return plsc.VectorSubcoreMesh(core_axis_name="c", subcore_axis_name="s")


NBUF_G = 5


def _make_gather(n_nodes, d_feat, n_edges):
    epw = n_edges // NW
    nch = epw // C
    ngrp = nch // NBUF_G

    scratch = [pltpu.VMEM((epw,), jnp.int32), pltpu.VMEM((epw,), jnp.int32)]
    scratch += [pltpu.VMEM((C, d_feat), jnp.float32) for _ in range(2 * NBUF_G)]
    scratch += [pltpu.SemaphoreType.DMA for _ in range(4 * NBUF_G)]

    @functools.partial(
        pl.kernel,
        mesh=_mesh(),
        out_type=[
            jax.ShapeDtypeStruct((n_edges, d_feat), jnp.float32),
            jax.ShapeDtypeStruct((n_edges, d_feat), jnp.float32),
        ],
        scratch_types=scratch,
    )
    def gather_k(coords, srcv, dstv, gs, gd, *scr):
        sidx, didx = scr[0], scr[1]
        rs = scr[2:2 + NBUF_G]
        rd = scr[2 + NBUF_G:2 + 2 * NBUF_G]
        base_sem = 2 + 2 * NBUF_G
        gsem_s = scr[base_sem:base_sem + NBUF_G]
        gsem_d = scr[base_sem + NBUF_G:base_sem + 2 * NBUF_G]
        wsem_s = scr[base_sem + 2 * NBUF_G:base_sem + 3 * NBUF_G]
        wsem_d = scr[base_sem + 3 * NBUF_G:base_sem + 4 * NBUF_G]

        cid = lax.axis_index("c")
        sid = lax.axis_index("s")
        wid = sid * NC + cid
        base = pl.multiple_of(wid * epw, 8)
        pltpu.sync_copy(srcv.at[pl.ds(base, epw)], sidx)
        pltpu.sync_copy(dstv.at[pl.ds(base, epw)], didx)

        def fire(ch, b):
            off = pl.multiple_of(ch * C, 8)
            pltpu.async_copy(coords.at[sidx.at[pl.ds(off, C)]], rs[b], gsem_s[b])
            pltpu.async_copy(coords.at[didx.at[pl.ds(off, C)]], rd[b], gsem_d[b])

        for b in range(NBUF_G):
            fire(b, b)

        def grp(g, carry):
            for b in range(NBUF_G):
                ch = g * NBUF_G + b
                off = pl.multiple_of(ch * C, 8)
                pltpu.make_async_copy(coords.at[pl.ds(0, C)], rs[b], gsem_s[b]).wait()
                pltpu.make_async_copy(coords.at[pl.ds(0, C)], rd[b], gsem_d[b]).wait()
                pltpu.async_copy(rs[b], gs.at[pl.ds(base + off, C)], wsem_s[b])
                pltpu.async_copy(rd[b], gd.at[pl.ds(base + off, C)], wsem_d[b])
                pltpu.make_async_copy(rs[b], gs.at[pl.ds(base, C)], wsem_s[b]).wait()
                pltpu.make_async_copy(rd[b], gd.at[pl.ds(base, C)], wsem_d[b]).wait()
                nxt = ch + NBUF_G

                @pl.when(nxt < nch)
                def _():
                    fire(nxt, b)
            return carry

        lax.fori_loop(0, ngrp, grp, 0)

    return gather_k


NBUF_S = 2


def _make_scatter(n_nodes, n_edges):
    epw = n_edges // NS
    nch = epw // C
    nzn = n_nodes // C

    scratch = [pltpu.VMEM((nch, C), jnp.int32)]
    scratch += [pltpu.VMEM((C, 128), jnp.float32) for _ in range(NBUF_S)]
    scratch += [pltpu.VMEM_SHARED((n_nodes, 128), jnp.float32),
                pltpu.VMEM_SHARED((NDPAD, 128), jnp.float32)]
    scratch += [pltpu.SemaphoreType.DMA for _ in range(2 * NBUF_S)]

    @functools.partial(
        pl.kernel,
        mesh=_mesh(),
        out_type=[
            jax.ShapeDtypeStruct((n_nodes, 128), jnp.float32),
            jax.ShapeDtypeStruct((NDPAD, 128), jnp.float32),
        ],
        scratch_types=scratch,
    )
    def scatter_k(ps, pd, dps, dpd, sidx3, didx3, sidx83, didx83,
                  outn, outd, *scr):
        idxb = scr[0]
        rb = scr[1:1 + NBUF_S]
        accn, accd = scr[1 + NBUF_S], scr[2 + NBUF_S]
        lsem = scr[3 + NBUF_S:3 + 2 * NBUF_S]
        ssem = scr[3 + 2 * NBUF_S:3 + 3 * NBUF_S]
        cid = lax.axis_index("c")
        sid = lax.axis_index("s")

        def zrow(r, carry):
            for kk in range(8):
                rb[0][r, pl.ds(kk * 16, 16)] = jnp.zeros((16,), jnp.float32)
            return carry

        lax.fori_loop(0, C, zrow, 0)

        def zc(jj, carry):
            ch = jj * NS + sid

            @pl.when(ch < nzn)
            def _():
                pltpu.sync_copy(rb[0], accn.at[pl.ds(pl.multiple_of(ch * C, 8), C)])
            return carry

        lax.fori_loop(0, (nzn + NS - 1) // NS, zc, 0)
        pltpu.sync_copy(rb[0], accd.at[pl.ds(pl.multiple_of(sid * C, 8), C)])
        plsc.subcore_barrier()

        def do_pass(payload, idx3, acc):
            pltpu.sync_copy(idx3.at[sid], idxb)
            pbase = pl.multiple_of(sid * epw, 8)

            def load(ch, b):
                off = pl.multiple_of(ch * C, 8)
                pltpu.async_copy(payload.at[pl.ds(pbase + off, C)], rb[b], lsem[b])

            for b in range(NBUF_S):
                load(b, b)

            def grp(g, carry2):
                for b in range(NBUF_S):
                    ch = g * NBUF_S + b
                    pltpu.make_async_copy(
                        payload.at[pl.ds(pbase, C)], rb[b], lsem[b]).wait()
                    pltpu.async_copy(
                        rb[b], acc.at[idxb.at[ch]], ssem[b], add=True)
                    pltpu.make_async_copy(
                        rb[b], acc.at[idxb.at[ch]], ssem[b]).wait()
                    nxt = ch + NBUF_S

                    @pl.when(nxt < nch)
                    def _():
                        load(nxt, b)
                return carry2

            lax.fori_loop(0, nch // NBUF_S, grp, 0)

        @pl.when(cid == 0)
        def _():
            do_pass(ps, sidx3, accn)
            do_pass(pd, didx3, accn)

        @pl.when(cid == 1)
        def _():
            do_pass(dps, sidx83, accd)
            do_pass(dpd, didx83, accd)

        plsc.subcore_barrier()

        @pl.when((sid == 0) & (cid == 0))
        def _():
            pltpu.sync_copy(accn, outn)

        @pl.when((sid == 0) & (cid == 1))
        def _():
            pltpu.sync_copy(accd, outd)

    return scatter_k


def _edge_body(gs_ref, gd_ref, ea_ref, nf_ref, src_ref, dst_ref,
               w1e_ref, w1s_ref, w1d_ref, b1_ref, w2v_ref, b2v_ref,
               w2s_ref, b2s_ref, ps_ref, pd_ref, dps_ref, dpd_ref):
    x = jnp.dot(ea_ref[...], w1e_ref[...], preferred_element_type=jnp.float32)
    x = x + jnp.dot(gs_ref[...], w1s_ref[...], preferred_element_type=jnp.float32)
    x = x + jnp.dot(gd_ref[...], w1d_ref[...], preferred_element_type=jnp.float32)
    h = jnp.maximum(x + b1_ref[...], 0.0)
    nf = nf_ref[...]
    v = (jnp.dot(h[:, :256], w2v_ref[...], preferred_element_type=jnp.float32)
         + b2v_ref[...]) * nf
    s = (jnp.dot(h[:, 256:], w2s_ref[...], preferred_element_type=jnp.float32)
         + b2s_ref[...]) * nf
    e = jnp.exp(s)
    b = v.shape[0]
    ps_ref[...] = jnp.concatenate(
        [v[:, 0:64] * e[:, 0:1], v[:, 64:128] * e[:, 1:2]], axis=1)
    pd_ref[...] = jnp.concatenate(
        [v[:, 128:192] * e[:, 2:3], v[:, 192:256] * e[:, 3:4]], axis=1)
    lane = lax.broadcasted_iota(jnp.int32, (b, 128), 1)
    scol = (src_ref[...] % 8) * 16
    dcol = (dst_ref[...] % 8) * 16
    zero = jnp.zeros((b, 128), jnp.float32)
    dps_ref[...] = (jnp.where(lane == scol, e[:, 0:1], zero)
                    + jnp.where(lane == scol + 8, e[:, 1:2], zero))
    dpd_ref[...] = (jnp.where(lane == dcol, e[:, 2:3], zero)
                    + jnp.where(lane == dcol + 8, e[:, 3:4], zero))


def _run_edge_mlp(gs, gd, ea, nf2, src2, dst2,
                  w1e, w1s, w1d, b1, w2v, b2v, w2s, b2s):
    n_edges, d_feat = gs.shape
    d_edge = ea.shape[1]
    nblk = n_edges // EB
    full = lambda shp: pl.BlockSpec(shp, lambda i: (0, 0))
    row = lambda w: pl.BlockSpec((EB, w), lambda i: (i, 0))
    return pl.pallas_call(
        _edge_body,
        grid=(nblk,),
        in_specs=[
            row(d_feat), row(d_feat), row(d_edge), row(1), row(1), row(1),
            full((d_edge, 512)), full((d_feat, 512)), full((d_feat, 512)),
            full((1, 512)), full((256, 256)), full((1, 256)),
            full((256, 8)), full((1, 8)),
        ],
        out_specs=[row(128), row(128), row(128), row(128)],
        out_shape=[jax.ShapeDtypeStruct((n_edges, 128), jnp.float32)] * 4,
        compiler_params=pltpu.CompilerParams(
            dimension_semantics=("arbitrary",)),
    )(gs, gd, ea, nf2, src2, dst2, w1e, w1s, w1d, b1, w2v, b2v, w2s, b2s)


def _psi_body(*refs):
    accn_refs = refs[:K]
    den_refs = refs[K:2 * K]
    nfa_ref, p1_ref, pb1_ref, p2_ref, pb2_ref, out_ref = refs[2 * K:]
    a = accn_refs[0][...]
    for r in accn_refs[1:]:
        a = a + r[...]
    den = den_refs[0][...]
    for r in den_refs[1:]:
        den = den + r[...]
    nfa = nfa_ref[...]
    num = a * nfa
    d0 = den[:, 0:1] * nfa + 1e-9
    d1 = den[:, 8:9] * nfa + 1e-9
    vp = jnp.concatenate([num[:, 0:64] / d0, num[:, 64:128] / d1], axis=1)
    h = jnp.maximum(
        jnp.dot(vp, p1_ref[...], preferred_element_type=jnp.float32) + pb1_ref[...],
        0.0)
    out_ref[...] = (jnp.dot(h, p2_ref[...], preferred_element_type=jnp.float32)
                    + pb2_ref[...]) * nfa


def _run_psi(accns, dens, nfa2, p1, pb1, p2, pb2):
    n_nodes = accns[0].shape[0]
    d_out = p2.shape[1]
    d_in = p1.shape[0]
    nblk = n_nodes // NB
    full = lambda shp: pl.BlockSpec(shp, lambda i: (0, 0))
    in_specs = ([pl.BlockSpec((NB, 128), lambda i: (i, 0))] * K
                + [pl.BlockSpec((NB, 16), lambda i: (i, 0))] * K
                + [pl.BlockSpec((NB, 1), lambda i: (i, 0)),
                   full((d_in, d_in)), full((1, d_in)), full((d_in, d_out)),
                   full((1, d_out))])
    return pl.pallas_call(
        _psi_body,
        grid=(nblk,),
        in_specs=in_specs,
        out_specs=pl.BlockSpec((NB, d_out), lambda i: (i, 0)),
        out_shape=jax.ShapeDtypeStruct((n_nodes, d_out), jnp.float32),
    )(*accns, *dens, nfa2, p1, pb1, p2, pb2)


def kernel(coordinates, edge_attr, src, dst, non_fictitious,
           non_fictitious_addresses, params):
    n_nodes, d_feat = coordinates.shape
    n_edges, d_edge = edge_attr.shape
    es = n_edges // K

    w1 = jnp.concatenate([params[n][0][0] for n in _VNAMES + _SNAMES], axis=1)
    b1 = jnp.concatenate([params[n][0][1] for n in _VNAMES + _SNAMES])[None, :]
    w1e = w1[:d_edge]
    w1s = w1[d_edge:d_edge + d_feat]
    w1d = w1[d_edge + d_feat:]
    w2v = jnp.zeros((256, 256), jnp.float32)
    for m, n in enumerate(_VNAMES):
        w2v = w2v.at[m * 64:(m + 1) * 64, m * 64:(m + 1) * 64].set(params[n][1][0])
    b2v = jnp.concatenate([params[n][1][1] for n in _VNAMES])[None, :]
    w2s = jnp.zeros((256, 8), jnp.float32)
    for m, n in enumerate(_SNAMES):
        w2s = w2s.at[m * 64:(m + 1) * 64, m].set(params[n][1][0][:, 0])
    b2s = jnp.zeros((1, 8), jnp.float32).at[0, :4].set(
        jnp.stack([params[n][1][1][0] for n in _SNAMES]))
    p1, pb1 = params["psi"][0]
    p2, pb2 = params["psi"][1]

    gather_fn = _make_gather(n_nodes, d_feat, es)
    scatter_fn = _make_scatter(n_nodes, es)
    nch_s = (es // NS) // C
    ishape = (NS, nch_s, C)
    src8 = src // 8
    dst8 = dst // 8

    accns, dens = [], []
    for k in range(K):
        sl = slice(k * es, (k + 1) * es)
        src_k, dst_k = src[sl], dst[sl]
        gs, gd = gather_fn(coordinates, src_k, dst_k)
        ps, pd, dps, dpd = _run_edge_mlp(
            gs, gd, edge_attr[sl], non_fictitious[sl][:, None],
            src_k[:, None], dst_k[:, None],
            w1e, w1s, w1d, b1, w2v, b2v, w2s, b2s)
        accn, accd = scatter_fn(
            ps, pd, dps, dpd,
            src_k.reshape(ishape), dst_k.reshape(ishape),
            src8[sl].reshape(ishape), dst8[sl].reshape(ishape))
        accns.append(accn)
        dens.append(accd.reshape(NDPAD * 8, 16)[:n_nodes])
    return _run_psi(accns, dens, non_fictitious_addresses[:, None],
                    p1, pb1[None, :], p2, pb2[None, :])
